# baseline (speedup 1.0000x reference)
import functools

import numpy as np
import jax
import jax.numpy as jnp
from jax import lax
from jax.experimental import pallas as pl
from jax.experimental.pallas import tpu as pltpu
from jax.experimental.pallas import tpu_sc as plsc

_B = 4096
_F = 26
_V = 100000
_D = 64
_NF = _F + 1

_NC, _NS = 2, 16
_NW = _NC * _NS
_ROWS = _F * _B
_CHUNK = 128
_NCHUNK = _ROWS // (_NW * _CHUNK)
_FPERROW = _B // _CHUNK


def _sc_gather(emb_flat, lsi2d):
    mesh = plsc.VectorSubcoreMesh(core_axis_name="c", subcore_axis_name="s")

    @functools.partial(
        pl.kernel,
        mesh=mesh,
        compiler_params=pltpu.CompilerParams(use_tc_tiling_on_sc=False),
        out_type=jax.ShapeDtypeStruct((_ROWS, _D), jnp.float32),
        scratch_types=[
            pltpu.VMEM((_NCHUNK, _CHUNK), jnp.int32),
            pltpu.VMEM((_CHUNK, _D), jnp.float32),
            pltpu.VMEM((_CHUNK, _D), jnp.float32),
            pltpu.SemaphoreType.DMA,
            pltpu.SemaphoreType.DMA,
        ],
    )
    def k(emb_hbm, lsi_hbm, out_hbm, idx_v, rows_a, rows_b, sem_a, sem_b):
        wid = lax.axis_index("s") * _NC + lax.axis_index("c")
        row0 = wid * _NCHUNK
        pltpu.sync_copy(lsi_hbm.at[wid], idx_v)
        for c in range(_NCHUNK):
            off = ((row0 + c) // _FPERROW) * _V
            for g in range(_CHUNK // 16):
                sl = (c, pl.ds(g * 16, 16))
                idx_v[sl] = idx_v[sl] + off
        bufs = ((rows_a, sem_a), (rows_b, sem_b))
        cps = []
        for c in range(_NCHUNK):
            buf, sem = bufs[c % 2]
            cps.append(pltpu.async_copy(emb_hbm.at[idx_v.at[c]], buf, sem))
            if c >= 1:
                pbuf, _ = bufs[(c - 1) % 2]
                cps[c - 1].wait()
                pltpu.sync_copy(
                    pbuf, out_hbm.at[pl.ds((row0 + c - 1) * _CHUNK, _CHUNK)]
                )
        cps[-1].wait()
        pltpu.sync_copy(
            bufs[(_NCHUNK - 1) % 2][0],
            out_hbm.at[pl.ds((row0 + _NCHUNK - 1) * _CHUNK, _CHUNK)],
        )

    return k(emb_flat, lsi2d)


_BB = 512
_NBLK = _B // _BB


def _tc_body(xref, lyref, wb0, bb0, wb1, bb1, wb2, bb2,
             wt0a, wt0p, bt0, wt1, bt1, wt2, bt2, oref):
    f32 = jnp.float32
    x = xref[...]
    x = jnp.maximum(jnp.dot(x, wb0[...], preferred_element_type=f32) + bb0[...], 0.0)
    x = jnp.maximum(jnp.dot(x, wb1[...], preferred_element_type=f32) + bb1[...], 0.0)
    x = jnp.maximum(jnp.dot(x, wb2[...], preferred_element_type=f32) + bb2[...], 0.0)
    ly = lyref[...]
    tt = jnp.concatenate([x[None], ly], axis=0)
    cols = [jnp.sum(tt * tt[i][None], axis=-1) for i in range(_NF)]
    zffm = jnp.concatenate(cols, axis=0)
    h = jnp.dot(x, wt0a[...], preferred_element_type=f32)
    h = h + lax.dot_general(zffm, wt0p[...], (((0,), (0,)), ((), ())),
                            preferred_element_type=f32)
    h = jnp.maximum(h + bt0[...], 0.0)
    h = jnp.maximum(jnp.dot(h, wt1[...], preferred_element_type=f32) + bt1[...], 0.0)
    z = jnp.dot(h, wt2[...], preferred_element_type=f32) + bt2[...]
    oref[...] = 1.0 / (1.0 + jnp.exp(-z))


def _full(shape):
    nd = len(shape)
    return pl.BlockSpec(shape, lambda i, _nd=nd: (0,) * _nd)


def kernel(dense_x, lS_i, W_emb, Wb0, bb0, Wb1, bb1, Wb2, bb2,
           Wt0, bt0, Wt1, bt1, Wt2, bt2):
    emb_flat = W_emb.reshape(_F * _V, _D)
    lsi3d = lS_i.reshape(_NW, _NCHUNK, _CHUNK)
    gathered = _sc_gather(emb_flat, lsi3d)
    ly = gathered.reshape(_F, _B, _D)
    pairs = np.array([i * _NF + j for i in range(_NF) for j in range(i)],
                     dtype=np.int32)
    wt0p = jnp.zeros((_NF * _NF, 512), jnp.float32).at[pairs].set(Wt0[:, 64:].T)
    out = pl.pallas_call(
        _tc_body,
        grid=(_NBLK,),
        in_specs=[
            pl.BlockSpec((_BB, 13), lambda i: (i, 0)),
            pl.BlockSpec((_F, _BB, _D), lambda i: (0, i, 0)),
            _full((13, 512)), _full((512,)),
            _full((512, 256)), _full((256,)),
            _full((256, 64)), _full((64,)),
            _full((64, 512)), _full((_NF * _NF, 512)), _full((512,)),
            _full((512, 256)), _full((256,)),
            _full((256, 1)), _full((1,)),
        ],
        out_specs=pl.BlockSpec((_BB, 1), lambda i: (i, 0)),
        out_shape=jax.ShapeDtypeStruct((_B, 1), jnp.float32),
    )(dense_x, ly, Wb0.T, bb0, Wb1.T, bb1, Wb2.T, bb2,
      Wt0[:, :64].T, wt0p, bt0, Wt1.T, bt1, Wt2.T, bt2)
    return out


# speedup vs baseline: 1.0777x; 1.0777x over previous
import functools

import numpy as np
import jax
import jax.numpy as jnp
from jax import lax
from jax.experimental import pallas as pl
from jax.experimental.pallas import tpu as pltpu
from jax.experimental.pallas import tpu_sc as plsc

_B = 4096
_F = 26
_V = 100000
_D = 64
_NF = _F + 1

_NC, _NS = 2, 16
_NW = _NC * _NS
_ROWS = _F * _B
_CHUNK = 128
_NCHUNK = _ROWS // (_NW * _CHUNK)
_FPERROW = _B // _CHUNK


def _sc_gather(emb_flat, lsi2d):
    mesh = plsc.VectorSubcoreMesh(core_axis_name="c", subcore_axis_name="s")

    @functools.partial(
        pl.kernel,
        mesh=mesh,
        compiler_params=pltpu.CompilerParams(use_tc_tiling_on_sc=False),
        out_type=jax.ShapeDtypeStruct((_ROWS, _D), jnp.float32),
        scratch_types=[
            pltpu.VMEM((_NCHUNK, _CHUNK), jnp.int32),
            pltpu.VMEM((_CHUNK, _D), jnp.float32),
            pltpu.VMEM((_CHUNK, _D), jnp.float32),
            pltpu.SemaphoreType.DMA,
            pltpu.SemaphoreType.DMA,
        ],
    )
    def k(emb_hbm, lsi_hbm, out_hbm, idx_v, rows_a, rows_b, sem_a, sem_b):
        wid = lax.axis_index("s") * _NC + lax.axis_index("c")
        row0 = wid * _NCHUNK
        pltpu.sync_copy(lsi_hbm.at[wid], idx_v)
        for c in range(_NCHUNK):
            off = ((row0 + c) // _FPERROW) * _V
            for g in range(_CHUNK // 16):
                sl = (c, pl.ds(g * 16, 16))
                idx_v[sl] = idx_v[sl] + off
        bufs = ((rows_a, sem_a), (rows_b, sem_b))
        cps = []
        for c in range(_NCHUNK):
            buf, sem = bufs[c % 2]
            cps.append(pltpu.async_copy(emb_hbm.at[idx_v.at[c]], buf, sem))
            if c >= 1:
                pbuf, _ = bufs[(c - 1) % 2]
                cps[c - 1].wait()
                pltpu.sync_copy(
                    pbuf, out_hbm.at[pl.ds((row0 + c - 1) * _CHUNK, _CHUNK)]
                )
        cps[-1].wait()
        pltpu.sync_copy(
            bufs[(_NCHUNK - 1) % 2][0],
            out_hbm.at[pl.ds((row0 + _NCHUNK - 1) * _CHUNK, _CHUNK)],
        )

    return k(emb_flat, lsi2d)


_BB = 512
_NBLK = _B // _BB
_NPAIR = _NF * (_NF - 1) // 2


def _tc_body(xref, lyref, wb0, bb0, wb1, bb1, wb2, bb2,
             wt0a, wt0b, bt0, wt1, bt1, wt2, bt2, oref):
    f32 = jnp.float32

    def mm(a, b):
        return jnp.dot(a, b, preferred_element_type=f32)

    x = xref[...]
    x = jnp.maximum(mm(wb0[...], x) + bb0[...], 0.0)
    x = jnp.maximum(mm(wb1[...], x) + bb1[...], 0.0)
    x = jnp.maximum(mm(wb2[...], x) + bb2[...], 0.0)
    tt = jnp.concatenate([x[None], lyref[...]], axis=0)
    parts = [jnp.sum(tt[:i] * tt[i:i + 1], axis=1) for i in range(1, _NF)]
    zf = jnp.concatenate(parts, axis=0)
    h = mm(wt0a[...], x) + lax.dot_general(
        wt0b[...], zf, (((1,), (0,)), ((), ())), preferred_element_type=f32)
    h = jnp.maximum(h + bt0[...], 0.0)
    h = jnp.maximum(mm(wt1[...], h) + bt1[...], 0.0)
    z = mm(wt2[...], h) + bt2[...]
    oref[...] = 1.0 / (1.0 + jnp.exp(-z))


def _full(shape):
    nd = len(shape)
    return pl.BlockSpec(shape, lambda i, _nd=nd: (0,) * _nd)


def kernel(dense_x, lS_i, W_emb, Wb0, bb0, Wb1, bb1, Wb2, bb2,
           Wt0, bt0, Wt1, bt1, Wt2, bt2):
    emb_flat = W_emb.reshape(_F * _V, _D)
    lsi3d = lS_i.reshape(_NW, _NCHUNK, _CHUNK)
    gathered = _sc_gather(emb_flat, lsi3d)
    lyt = jnp.transpose(gathered.reshape(_F, _B, _D), (0, 2, 1))
    out = pl.pallas_call(
        _tc_body,
        grid=(_NBLK,),
        in_specs=[
            pl.BlockSpec((13, _BB), lambda i: (0, i)),
            pl.BlockSpec((_F, _D, _BB), lambda i: (0, 0, i)),
            _full((512, 13)), _full((512, 1)),
            _full((256, 512)), _full((256, 1)),
            _full((64, 256)), _full((64, 1)),
            _full((512, 64)), _full((512, _NPAIR)), _full((512, 1)),
            _full((256, 512)), _full((256, 1)),
            _full((1, 256)), _full((1, 1)),
        ],
        out_specs=pl.BlockSpec((1, _BB), lambda i: (0, i)),
        out_shape=jax.ShapeDtypeStruct((1, _B), jnp.float32),
    )(dense_x.T, lyt, Wb0, bb0[:, None], Wb1, bb1[:, None], Wb2, bb2[:, None],
      Wt0[:, :64], Wt0[:, 64:], bt0[:, None], Wt1, bt1[:, None],
      Wt2, bt2[:, None])
    return out.reshape(_B, 1)
